# baseline (speedup 1.0000x reference)
import functools

import jax
import jax.numpy as jnp
from jax import lax
from jax.experimental import pallas as pl
from jax.experimental.pallas import tpu as pltpu
from jax.experimental.pallas import tpu_sc as plsc

B_BLK = 128


def _add_body(x_ref, emb_ref, o_ref):
    o_ref[...] = x_ref[...] + emb_ref[...][:, None, :]


def _make_sc_gather(V, D, B):
    info = plsc.get_sparse_core_info()
    NC, NS = info.num_cores, info.num_subcores
    NW = NC * NS
    assert B % (8 * NW) == 0
    b_per_w = B // NW
    mesh = plsc.VectorSubcoreMesh(core_axis_name="c", subcore_axis_name="s")

    @functools.partial(
        pl.kernel,
        mesh=mesh,
        out_type=jax.ShapeDtypeStruct((B, D), jnp.float32),
        scratch_types=[
            pltpu.VMEM((b_per_w,), jnp.int32),
            pltpu.VMEM((b_per_w, D), jnp.float32),
            pltpu.SemaphoreType.DMA,
        ],
    )
    def sc_gather(table_hbm, idx_hbm, out_hbm, idx_v, rows_v, sem):
        wid = lax.axis_index("s") * NC + lax.axis_index("c")
        base = wid * b_per_w
        pltpu.sync_copy(idx_hbm.at[pl.ds(base, b_per_w)], idx_v)
        pltpu.async_copy(table_hbm.at[idx_v], rows_v, sem).wait()
        pltpu.sync_copy(rows_v, out_hbm.at[pl.ds(base, b_per_w)])

    return sc_gather


def kernel(x, g_id, embedding):
    B, L, H = x.shape
    V = embedding.shape[0]
    emb = _make_sc_gather(V, H, B)(embedding, g_id.astype(jnp.int32))
    nb = B // B_BLK
    return pl.pallas_call(
        _add_body,
        grid=(nb,),
        in_specs=[
            pl.BlockSpec((B_BLK, L, H), lambda i: (i, 0, 0)),
            pl.BlockSpec((B_BLK, H), lambda i: (i, 0)),
        ],
        out_specs=pl.BlockSpec((B_BLK, L, H), lambda i: (i, 0, 0)),
        out_shape=jax.ShapeDtypeStruct((B, L, H), x.dtype),
    )(x, emb)
